# Initial kernel scaffold; baseline (speedup 1.0000x reference)
import jax
import jax.numpy as jnp
from jax.experimental import pallas as pl


def kernel(hidden_tensor, weight, scores_bias):
    raise NotImplementedError("write your pallas kernel here")


# speedup vs baseline: 5.2656x; 5.2656x over previous
import jax
import jax.numpy as jnp
from jax.experimental import pallas as pl

_B = 1024

_TOPK_SCALE = 2.5


def _router_block(x_ref, w_ref, b_ref, idx_ref, wgt_ref):
    x = x_ref[...]
    w = w_ref[...]
    bias = b_ref[...]

    h = jax.lax.dot_general(w, x, (((1,), (1,)), ((), ())),
                            preferred_element_type=jnp.float32)
    s = jax.nn.sigmoid(h) + bias

    B = s.shape[1]
    iota = jax.lax.broadcasted_iota(jnp.int32, (8, B), 0)
    gidx = iota >> 1

    p = jnp.concatenate(
        [s[1:2], s[0:1], s[3:4], s[2:3], s[5:6], s[4:5], s[7:8], s[6:7]],
        axis=0)
    gw = s + p

    NEG = jnp.float32(-1.0)

    g1v = jnp.max(gw, axis=0, keepdims=True)
    g1 = jnp.min(jnp.where(gw == g1v, gidx, 4), axis=0, keepdims=True)
    gw2 = jnp.where(gidx == g1, NEG, gw)
    g2v = jnp.max(gw2, axis=0, keepdims=True)
    g2 = jnp.min(jnp.where(gw2 == g2v, gidx, 4), axis=0, keepdims=True)
    sel = (gidx == g1) | (gidx == g2)

    m = jnp.where(sel, s, 0.0)
    v1 = jnp.max(m, axis=0, keepdims=True)
    e1 = jnp.min(jnp.where(m == v1, iota, 8), axis=0, keepdims=True)
    m2 = jnp.where(iota == e1, NEG, m)
    v2 = jnp.max(m2, axis=0, keepdims=True)
    e2 = jnp.min(jnp.where(m2 == v2, iota, 8), axis=0, keepdims=True)

    scale = _TOPK_SCALE / (v1 + v2)
    idx_ref[...] = jnp.concatenate([e1, e2], axis=0)
    wgt_ref[...] = jnp.concatenate([v1 * scale, v2 * scale], axis=0)


def kernel(hidden_tensor, weight, scores_bias):
    T, D = hidden_tensor.shape
    E = weight.shape[0]
    idx_t, wgt_t = pl.pallas_call(
        _router_block,
        grid=(T // _B,),
        in_specs=[
            pl.BlockSpec((_B, D), lambda i: (i, 0)),
            pl.BlockSpec((E, D), lambda i: (0, 0)),
            pl.BlockSpec((E, 1), lambda i: (0, 0)),
        ],
        out_specs=[
            pl.BlockSpec((2, _B), lambda i: (0, i)),
            pl.BlockSpec((2, _B), lambda i: (0, i)),
        ],
        out_shape=[
            jax.ShapeDtypeStruct((2, T), jnp.int32),
            jax.ShapeDtypeStruct((2, T), jnp.float32),
        ],
    )(hidden_tensor, weight, scores_bias.reshape(E, 1))
    return (idx_t.T, wgt_t.T)
